# baseline (speedup 1.0000x reference)
import jax
import jax.numpy as jnp
from jax.experimental import pallas as pl

N_NODES = 10000


def _bias_add_kernel(x_ref, b_ref, o_ref):
    o_ref[...] = x_ref[...] + b_ref[...]


def _bias_add(x, b):
    return pl.pallas_call(
        _bias_add_kernel,
        out_shape=jax.ShapeDtypeStruct(x.shape, x.dtype),
    )(x, b[None, :])


def _gcn(x, W, b, row, col, ew, n_nodes):
    x = x @ W
    loop = jnp.arange(n_nodes, dtype=row.dtype)
    row2 = jnp.concatenate([row, loop])
    col2 = jnp.concatenate([col, loop])
    ew2 = jnp.concatenate([ew, jnp.ones((n_nodes,), dtype=ew.dtype)])
    deg = jax.ops.segment_sum(ew2, col2, num_segments=n_nodes)
    deg_inv_sqrt = jnp.where(deg > 0, jax.lax.rsqrt(jnp.maximum(deg, 1e-12)), 0.0)
    norm = deg_inv_sqrt[row2] * ew2 * deg_inv_sqrt[col2]
    msg = x[row2] * norm[:, None]
    out = jax.ops.segment_sum(msg, col2, num_segments=n_nodes)
    return _bias_add(out, b)


def kernel(z, edge_index, edge_attr, W1, b1, W2, b2):
    row = edge_index[0]
    col = edge_index[1]
    hidden = jax.nn.relu(_gcn(z, W1, b1, row, col, edge_attr, N_NODES))
    return _gcn(hidden, W2, b2, row, col, edge_attr, N_NODES)


# speedup vs baseline: 8.2938x; 8.2938x over previous
import functools

import jax
import jax.numpy as jnp
from jax import lax
from jax.experimental import pallas as pl
from jax.experimental.pallas import tpu as pltpu
from jax.experimental.pallas import tpu_sc as plsc

N_NODES = 10000
N_EDGES = 320000
F = 128

NC = 2
NS = 16
NW = NC * NS
C = 128
E_PAD = 327680
EPW = E_PAD // NW
NCH = EPW // C
N_PAD = 10240
RPS = N_PAD // NS
ZR = 128
B_E = 2560

_mesh = plsc.VectorSubcoreMesh(core_axis_name="c", subcore_axis_name="s")
_sc_params = pltpu.CompilerParams(needs_layout_passes=False)


@functools.partial(
    pl.kernel,
    out_type=jax.ShapeDtypeStruct((NW * N_PAD,), jnp.float32),
    mesh=_mesh,
    compiler_params=_sc_params,
    scratch_types=[
        pltpu.VMEM((N_PAD,), jnp.float32),
        pltpu.VMEM((EPW,), jnp.int32),
        pltpu.VMEM((EPW,), jnp.float32),
    ],
)
def _deg_kernel(col_hbm, ew_hbm, out_hbm, deg_v, col_v, ew_v):
    wid = lax.axis_index("c") * NS + lax.axis_index("s")

    @pl.loop(0, N_PAD, step=16)
    def _(i):
        deg_v[pl.ds(i, 16)] = jnp.zeros((16,), jnp.float32)

    pltpu.sync_copy(col_hbm.at[pl.ds(wid * EPW, EPW)], col_v)
    pltpu.sync_copy(ew_hbm.at[pl.ds(wid * EPW, EPW)], ew_v)

    @pl.loop(0, EPW, step=16)
    def _(i):
        idx = col_v[pl.ds(i, 16)]
        val = ew_v[pl.ds(i, 16)]
        plsc.addupdate_scatter(deg_v, [idx], val)

    pltpu.sync_copy(deg_v, out_hbm.at[pl.ds(wid * N_PAD, N_PAD)])


@functools.partial(
    pl.kernel,
    out_type=jax.ShapeDtypeStruct((NC, N_PAD, F), jnp.float32),
    mesh=_mesh,
    compiler_params=_sc_params,
    scratch_types=[
        pltpu.VMEM_SHARED((N_PAD, F), jnp.float32),
        pltpu.VMEM((B_E,), jnp.int32),
        pltpu.VMEM((B_E,), jnp.int32),
        pltpu.VMEM((B_E,), jnp.float32),
        pltpu.VMEM((C, F), jnp.float32),
    ],
)
def _agg_kernel(y_hbm, row_hbm, col_hbm, ew_hbm, out_hbm,
                acc, row_v, col_v, ew_v, gbuf):
    c = lax.axis_index("c")
    s = lax.axis_index("s")
    wid = c * NS + s

    @pl.loop(0, C)
    def _(r):
        for j in range(F // 16):
            gbuf[r, pl.ds(j * 16, 16)] = jnp.zeros((16,), jnp.float32)

    @pl.loop(0, RPS // ZR)
    def _(k):
        pltpu.sync_copy(gbuf, acc.at[pl.ds(s * RPS + k * ZR, ZR)])

    plsc.subcore_barrier()

    @pl.loop(0, EPW // B_E)
    def _(bk):
        base = wid * EPW + bk * B_E
        pltpu.sync_copy(row_hbm.at[pl.ds(base, B_E)], row_v)
        pltpu.sync_copy(col_hbm.at[pl.ds(base, B_E)], col_v)
        pltpu.sync_copy(ew_hbm.at[pl.ds(base, B_E)], ew_v)

        @pl.loop(0, B_E // C)
        def _(ch):
            off = ch * C
            pltpu.sync_copy(y_hbm.at[row_v.at[pl.ds(off, C)]], gbuf)

            @pl.loop(0, C, step=16)
            def _(e0):
                ewv = ew_v[pl.ds(off + e0, 16)]
                for i in range(16):
                    wv = jnp.full((16,), ewv[i], jnp.float32)
                    for j in range(F // 16):
                        sl = pl.ds(j * 16, 16)
                        gbuf[e0 + i, sl] = gbuf[e0 + i, sl] * wv

            pltpu.sync_copy(gbuf, acc.at[col_v.at[pl.ds(off, C)]], add=True)

    plsc.subcore_barrier()

    @pl.loop(0, RPS // ZR)
    def _(k):
        base = s * RPS + k * ZR
        pltpu.sync_copy(acc.at[pl.ds(base, ZR)], out_hbm.at[c].at[pl.ds(base, ZR)])


def _prep_body(degp_ref, z_ref, w1_ref, b1_ref, dinv_ref, y1_ref, s1_ref):
    deg = jnp.sum(degp_ref[...], axis=0)[:N_NODES] + 1.0
    dinv = lax.rsqrt(deg)
    xw = jnp.dot(z_ref[...], w1_ref[...], preferred_element_type=jnp.float32)
    di = dinv[:, None]
    dinv_ref[...] = di
    y1_ref[...] = xw * di
    s1_ref[...] = xw * (di * di) + b1_ref[...]


def _mid_body(p_ref, s1_ref, dinv_ref, w2_ref, b2_ref, y2_ref, s2_ref):
    di = dinv_ref[...]
    agg = (p_ref[0] + p_ref[1])[:N_NODES]
    h = jnp.maximum(di * agg + s1_ref[...], 0.0)
    xw = jnp.dot(h, w2_ref[...], preferred_element_type=jnp.float32)
    y2_ref[...] = xw * di
    s2_ref[...] = xw * (di * di) + b2_ref[...]


def _final_body(p_ref, s2_ref, dinv_ref, o_ref):
    di = dinv_ref[...]
    agg = (p_ref[0] + p_ref[1])[:N_NODES]
    o_ref[...] = di * agg + s2_ref[...]


def _prep(degp, z, W1, b1):
    return pl.pallas_call(
        _prep_body,
        out_shape=[
            jax.ShapeDtypeStruct((N_NODES, 1), jnp.float32),
            jax.ShapeDtypeStruct((N_NODES, F), jnp.float32),
            jax.ShapeDtypeStruct((N_NODES, F), jnp.float32),
        ],
    )(degp, z, W1, b1)


def _mid(p1, s1, dinv, W2, b2):
    return pl.pallas_call(
        _mid_body,
        out_shape=[
            jax.ShapeDtypeStruct((N_NODES, F), jnp.float32),
            jax.ShapeDtypeStruct((N_NODES, F), jnp.float32),
        ],
    )(p1, s1, dinv, W2, b2)


def _final(p2, s2, dinv):
    return pl.pallas_call(
        _final_body,
        out_shape=jax.ShapeDtypeStruct((N_NODES, F), jnp.float32),
    )(p2, s2, dinv)


def kernel(z, edge_index, edge_attr, W1, b1, W2, b2):
    row = edge_index[0].astype(jnp.int32)
    col = edge_index[1].astype(jnp.int32)
    ew = edge_attr.astype(jnp.float32)
    pad = E_PAD - N_EDGES
    row1 = jnp.concatenate([row, jnp.zeros((pad,), jnp.int32)])
    col1 = jnp.concatenate([col, jnp.zeros((pad,), jnp.int32)])
    ew1 = jnp.concatenate([ew, jnp.zeros((pad,), jnp.float32)])

    degp = _deg_kernel(col1, ew1).reshape(NW, N_PAD)
    dinv, y1, s1 = _prep(degp, z, W1, b1)
    p1 = _agg_kernel(y1, row1, col1, ew1)
    y2, s2 = _mid(p1, s1, dinv, W2, b2)
    p2 = _agg_kernel(y2, row1, col1, ew1)
    return _final(p2, s2, dinv)
